# Initial kernel scaffold; baseline (speedup 1.0000x reference)
import jax
import jax.numpy as jnp
from jax.experimental import pallas as pl


def kernel(x, edge_index):
    raise NotImplementedError("write your pallas kernel here")


# speedup vs baseline: 6.0030x; 6.0030x over previous
import functools

import jax
import jax.numpy as jnp
from jax import lax
from jax.experimental import pallas as pl
from jax.experimental.pallas import tpu as pltpu
from jax.experimental.pallas import tpu_sc as plsc

N = 10000
D = 128
DH = D // 2
E = 320000
NC, NS = 2, 16
EPT = E // NS
K = 80
NCHUNK = EPT // K
RPT = N // NS
DW = 16

_mesh = plsc.VectorSubcoreMesh(core_axis_name="c", subcore_axis_name="s")


@functools.partial(
    pl.kernel,
    mesh=_mesh,
    compiler_params=pltpu.CompilerParams(use_tc_tiling_on_sc=False),
    out_type=[
        jax.ShapeDtypeStruct((NC, NS, RPT, DH), jnp.float32),
        jax.ShapeDtypeStruct((NS, RPT, DW), jnp.float32),
    ],
    scratch_types=[
        pltpu.VMEM((NCHUNK, K), jnp.int32),
        pltpu.VMEM((NCHUNK, K), jnp.int32),
        pltpu.VMEM((K, DH), jnp.float32),
        pltpu.VMEM((K, DW), jnp.float32),
        pltpu.VMEM_SHARED((N, DH), jnp.float32),
        pltpu.VMEM_SHARED((N, DW), jnp.float32),
    ],
)
def _sc_scatter(x_hbm, src_hbm, dst_hbm, zagg_hbm, zdeg_hbm, ones_hbm,
                agg_out, deg_out, src_v, dst_v, rows_v, ones_v, agg_s, deg_s):
    c = lax.axis_index("c")
    s = lax.axis_index("s")

    pltpu.sync_copy(zagg_hbm, agg_s.at[pl.ds(s * RPT, RPT)])
    pltpu.sync_copy(zdeg_hbm, deg_s.at[pl.ds(s * RPT, RPT)])
    pltpu.sync_copy(ones_hbm, ones_v)
    pltpu.sync_copy(src_hbm.at[s], src_v)
    pltpu.sync_copy(dst_hbm.at[s], dst_v)
    plsc.subcore_barrier()

    def body(i, carry):
        pltpu.sync_copy(x_hbm.at[c].at[src_v.at[i]], rows_v)
        pltpu.sync_copy(rows_v, agg_s.at[dst_v.at[i]], add=True)

        @pl.when(c == 0)
        def _():
            pltpu.sync_copy(ones_v, deg_s.at[dst_v.at[i]], add=True)

        return carry

    lax.fori_loop(0, NCHUNK, body, 0)
    plsc.subcore_barrier()

    pltpu.sync_copy(agg_s.at[pl.ds(s * RPT, RPT)], agg_out.at[c, s])

    @pl.when(c == 0)
    def _():
        pltpu.sync_copy(deg_s.at[pl.ds(s * RPT, RPT)], deg_out.at[s])


_TC_ROWS = 1000


def _tc_combine(agg_ref, deg_ref, y_ref):
    agg = jnp.concatenate([agg_ref[0], agg_ref[1]], axis=1)
    deg = deg_ref[:, 0:1]
    y_ref[...] = agg / jnp.maximum(deg, 1.0)


def kernel(x, edge_index):
    ei = edge_index.astype(jnp.int32)
    src = ei[0].reshape(NS, NCHUNK, K)
    dst = ei[1].reshape(NS, NCHUNK, K)
    x_halves = jnp.stack([x[:, :DH], x[:, DH:]])
    zagg = jnp.zeros((RPT, DH), jnp.float32)
    zdeg = jnp.zeros((RPT, DW), jnp.float32)
    ones = jnp.ones((K, DW), jnp.float32)

    agg_p, deg_p = _sc_scatter(x_halves, src, dst, zagg, zdeg, ones)
    agg_p = agg_p.reshape(NC, N, DH)
    deg_p = deg_p.reshape(N, DW)

    y = pl.pallas_call(
        _tc_combine,
        grid=(N // _TC_ROWS,),
        in_specs=[
            pl.BlockSpec((NC, _TC_ROWS, DH), lambda i: (0, i, 0)),
            pl.BlockSpec((_TC_ROWS, DW), lambda i: (i, 0)),
        ],
        out_specs=pl.BlockSpec((_TC_ROWS, D), lambda i: (i, 0)),
        out_shape=jax.ShapeDtypeStruct((N, D), jnp.float32),
    )(agg_p, deg_p)
    return y


# speedup vs baseline: 13.0114x; 2.1675x over previous
import functools

import jax
import jax.numpy as jnp
from jax import lax
from jax.experimental import pallas as pl
from jax.experimental.pallas import tpu as pltpu
from jax.experimental.pallas import tpu_sc as plsc

N = 10000
D = 128
DH = D // 2
E = 320000
NC, NS = 2, 16
EPT = E // NS
K = 80
NCHUNK = EPT // K
HCHUNK = NCHUNK // 2
RPT = N // NS
DW = 16
NB = 4

_mesh = plsc.VectorSubcoreMesh(core_axis_name="c", subcore_axis_name="s")


@functools.partial(
    pl.kernel,
    mesh=_mesh,
    compiler_params=pltpu.CompilerParams(use_tc_tiling_on_sc=False),
    out_type=[
        jax.ShapeDtypeStruct((NC, NS, RPT, DH), jnp.float32),
        jax.ShapeDtypeStruct((NC, NS, RPT, DW), jnp.float32),
    ],
    scratch_types=[
        pltpu.VMEM((NCHUNK, K), jnp.int32),
        pltpu.VMEM((NCHUNK, K), jnp.int32),
        pltpu.VMEM((NB, K, DH), jnp.float32),
        pltpu.VMEM((K, DW), jnp.float32),
        pltpu.VMEM_SHARED((N, DH), jnp.float32),
        pltpu.VMEM_SHARED((N, DW), jnp.float32),
        pltpu.SemaphoreType.DMA,
        pltpu.SemaphoreType.DMA,
        pltpu.SemaphoreType.DMA,
    ],
)
def _sc_scatter(x_hbm, src_hbm, dst_hbm, zagg_hbm, zdeg_hbm, ones_hbm,
                agg_out, deg_out, src_v, dst_v, rows_v, ones_v, agg_s, deg_s,
                gsem, ssem, dsem):
    c = lax.axis_index("c")
    s = lax.axis_index("s")

    pltpu.sync_copy(zagg_hbm, agg_s.at[pl.ds(s * RPT, RPT)])
    pltpu.sync_copy(zdeg_hbm, deg_s.at[pl.ds(s * RPT, RPT)])
    pltpu.sync_copy(ones_hbm, ones_v)
    pltpu.sync_copy(src_hbm.at[s], src_v)
    pltpu.sync_copy(dst_hbm.at[s], dst_v)
    plsc.subcore_barrier()

    for b in range(NB - 1):
        pltpu.async_copy(x_hbm.at[c].at[src_v.at[b]], rows_v.at[b], gsem)

    def body(i, carry):
        b = lax.rem(i, NB)

        @pl.when(i >= 1)
        def _():
            pltpu.make_async_copy(rows_v.at[0], agg_s.at[dst_v.at[0]],
                                  ssem).wait()

        @pl.when(i + NB - 1 < NCHUNK)
        def _():
            nxt = i + NB - 1
            pltpu.async_copy(x_hbm.at[c].at[src_v.at[nxt]],
                             rows_v.at[lax.rem(nxt, NB)], gsem)

        pltpu.make_async_copy(x_hbm.at[c].at[src_v.at[i]], rows_v.at[b],
                              gsem).wait()
        pltpu.async_copy(rows_v.at[b], agg_s.at[dst_v.at[i]], ssem, add=True)

        j = i - c * HCHUNK

        @pl.when((1 <= j) & (j < HCHUNK))
        def _():
            pltpu.make_async_copy(ones_v, deg_s.at[dst_v.at[0]], dsem).wait()

        @pl.when((0 <= j) & (j < HCHUNK))
        def _():
            pltpu.async_copy(ones_v, deg_s.at[dst_v.at[i]], dsem, add=True)

        return carry

    lax.fori_loop(0, NCHUNK, body, 0)

    pltpu.make_async_copy(rows_v.at[0], agg_s.at[dst_v.at[0]], ssem).wait()
    pltpu.make_async_copy(ones_v, deg_s.at[dst_v.at[0]], dsem).wait()

    plsc.subcore_barrier()

    pltpu.sync_copy(agg_s.at[pl.ds(s * RPT, RPT)], agg_out.at[c, s])
    pltpu.sync_copy(deg_s.at[pl.ds(s * RPT, RPT)], deg_out.at[c, s])


_TC_ROWS = 1000


def _tc_combine(agg_ref, deg_ref, y_ref):
    agg = jnp.concatenate([agg_ref[0], agg_ref[1]], axis=1)
    deg = deg_ref[0, :, 0:1] + deg_ref[1, :, 0:1]
    y_ref[...] = agg / jnp.maximum(deg, 1.0)


def kernel(x, edge_index):
    ei = edge_index.astype(jnp.int32)
    src = ei[0].reshape(NS, NCHUNK, K)
    dst = ei[1].reshape(NS, NCHUNK, K)
    x_halves = jnp.stack([x[:, :DH], x[:, DH:]])
    zagg = jnp.zeros((RPT, DH), jnp.float32)
    zdeg = jnp.zeros((RPT, DW), jnp.float32)
    ones = jnp.ones((K, DW), jnp.float32)

    agg_p, deg_p = _sc_scatter(x_halves, src, dst, zagg, zdeg, ones)
    agg_p = agg_p.reshape(NC, N, DH)
    deg_p = deg_p.reshape(NC, N, DW)

    y = pl.pallas_call(
        _tc_combine,
        grid=(N // _TC_ROWS,),
        in_specs=[
            pl.BlockSpec((NC, _TC_ROWS, DH), lambda i: (0, i, 0)),
            pl.BlockSpec((NC, _TC_ROWS, DW), lambda i: (0, i, 0)),
        ],
        out_specs=pl.BlockSpec((_TC_ROWS, D), lambda i: (i, 0)),
        out_shape=jax.ShapeDtypeStruct((N, D), jnp.float32),
    )(agg_p, deg_p)
    return y


# speedup vs baseline: 14.2454x; 1.0948x over previous
import functools

import jax
import jax.numpy as jnp
from jax import lax
from jax.experimental import pallas as pl
from jax.experimental.pallas import tpu as pltpu
from jax.experimental.pallas import tpu_sc as plsc

N = 10000
D = 128
DH = D // 2
E = 320000
NC, NS = 2, 16
EPT = E // NS
K = 80
NCHUNK = EPT // K
RPT = N // NS
RPB = 125
DW = 16
NB = 4
L = 16

_mesh = plsc.VectorSubcoreMesh(core_axis_name="c", subcore_axis_name="s")


@functools.partial(
    pl.kernel,
    mesh=_mesh,
    compiler_params=pltpu.CompilerParams(use_tc_tiling_on_sc=False),
    out_type=jax.ShapeDtypeStruct((N, D), jnp.float32),
    scratch_types=[
        pltpu.VMEM((NCHUNK, K), jnp.int32),
        pltpu.VMEM((NCHUNK, K), jnp.int32),
        pltpu.VMEM((NB, K, DH), jnp.float32),
        pltpu.VMEM((K, DW), jnp.float32),
        pltpu.VMEM((RPB, DH), jnp.float32),
        pltpu.VMEM((RPB, DW), jnp.float32),
        pltpu.VMEM_SHARED((N, DH), jnp.float32),
        pltpu.VMEM_SHARED((N, DW), jnp.float32),
        pltpu.SemaphoreType.DMA,
        pltpu.SemaphoreType.DMA,
        pltpu.SemaphoreType.DMA,
    ],
)
def _sc_smooth(x_hbm, src_hbm, dst_hbm, zagg_hbm, zdeg_hbm, ones_hbm,
               y_hbm, src_v, dst_v, rows_v, ones_v, agg_v, deg_v,
               agg_s, deg_s, gsem, ssem, dsem):
    c = lax.axis_index("c")
    s = lax.axis_index("s")
    xh = x_hbm.at[c]

    pltpu.sync_copy(zagg_hbm, agg_s.at[pl.ds(s * RPT, RPT)])
    pltpu.sync_copy(zdeg_hbm, deg_s.at[pl.ds(s * RPT, RPT)])
    pltpu.sync_copy(ones_hbm, ones_v)
    pltpu.sync_copy(src_hbm.at[s], src_v)
    pltpu.sync_copy(dst_hbm.at[s], dst_v)
    plsc.subcore_barrier()

    for b in range(NB - 1):
        pltpu.async_copy(xh.at[src_v.at[b]], rows_v.at[b], gsem)

    def body(i, carry):
        b = lax.rem(i, NB)

        @pl.when(i >= 1)
        def _():
            pltpu.make_async_copy(rows_v.at[0], agg_s.at[dst_v.at[0]],
                                  ssem).wait()
            pltpu.make_async_copy(ones_v, deg_s.at[dst_v.at[0]], dsem).wait()

        @pl.when(i + NB - 1 < NCHUNK)
        def _():
            nxt = i + NB - 1
            pltpu.async_copy(xh.at[src_v.at[nxt]], rows_v.at[lax.rem(nxt, NB)],
                             gsem)

        pltpu.make_async_copy(xh.at[src_v.at[i]], rows_v.at[b], gsem).wait()
        pltpu.async_copy(rows_v.at[b], agg_s.at[dst_v.at[i]], ssem, add=True)
        pltpu.async_copy(ones_v, deg_s.at[dst_v.at[i]], dsem, add=True)
        return carry

    lax.fori_loop(0, NCHUNK, body, 0)

    pltpu.make_async_copy(rows_v.at[0], agg_s.at[dst_v.at[0]], ssem).wait()
    pltpu.make_async_copy(ones_v, deg_s.at[dst_v.at[0]], dsem).wait()

    plsc.subcore_barrier()

    for p in range(RPT // RPB):
        row0 = s * RPT + p * RPB
        pltpu.sync_copy(agg_s.at[pl.ds(row0, RPB)], agg_v)
        pltpu.sync_copy(deg_s.at[pl.ds(row0, RPB)], deg_v)

        def norm(r, carry):
            inv = 1.0 / jnp.maximum(deg_v[r, :], 1.0)
            for q in range(DH // L):
                agg_v[r, pl.ds(q * L, L)] = agg_v[r, pl.ds(q * L, L)] * inv
            return carry

        lax.fori_loop(0, RPB, norm, 0)
        pltpu.sync_copy(agg_v,
                        y_hbm.at[pl.ds(row0, RPB), pl.ds(c * DH, DH)])


def kernel(x, edge_index):
    ei = edge_index.astype(jnp.int32)
    src = ei[0].reshape(NS, NCHUNK, K)
    dst = ei[1].reshape(NS, NCHUNK, K)
    x_halves = jnp.stack([x[:, :DH], x[:, DH:]])
    zagg = jnp.zeros((RPT, DH), jnp.float32)
    zdeg = jnp.zeros((RPT, DW), jnp.float32)
    ones = jnp.ones((K, DW), jnp.float32)
    return _sc_smooth(x_halves, src, dst, zagg, zdeg, ones)


# speedup vs baseline: 14.7197x; 1.0333x over previous
import functools

import jax
import jax.numpy as jnp
from jax import lax
from jax.experimental import pallas as pl
from jax.experimental.pallas import tpu as pltpu
from jax.experimental.pallas import tpu_sc as plsc

N = 10000
D = 128
DH = D // 2
E = 320000
NC, NS = 2, 16
EPT = E // NS
K = 80
NCHUNK = EPT // K
NB = 4
L = 16
HR = 640
ZR = 208

_mesh = plsc.VectorSubcoreMesh(core_axis_name="c", subcore_axis_name="s")


@functools.partial(
    pl.kernel,
    mesh=_mesh,
    compiler_params=pltpu.CompilerParams(use_tc_tiling_on_sc=False,
                                         needs_layout_passes=False),
    out_type=jax.ShapeDtypeStruct((N, D), jnp.float32),
    scratch_types=[
        pltpu.VMEM((NCHUNK, K), jnp.int32),
        pltpu.VMEM((NCHUNK, K), jnp.int32),
        pltpu.VMEM((NB, K, DH), jnp.float32),
        pltpu.VMEM((ZR, DH), jnp.float32),
        pltpu.VMEM((HR, L), jnp.float32),
        pltpu.VMEM((HR // L, L), jnp.float32),
        pltpu.VMEM((HR // 128, 128), jnp.int32),
        pltpu.VMEM_SHARED((N, DH), jnp.float32),
        pltpu.VMEM_SHARED((HR, L), jnp.float32),
        pltpu.SemaphoreType.DMA,
        pltpu.SemaphoreType.DMA,
    ],
)
def _sc_smooth(x_hbm, src_hbm, dst_hbm, idn_hbm,
               y_hbm, src_v, dst_v, rows_v, agg_v, hist_v, degm_v, idn_v,
               agg_s, deg_s, gsem, ssem):
    c = lax.axis_index("c")
    s = lax.axis_index("s")
    xh = x_hbm.at[c]
    ones = jnp.full((L,), 1.0, jnp.float32)
    zeros = jnp.zeros((L,), jnp.float32)

    pltpu.sync_copy(src_hbm.at[s], src_v)
    pltpu.sync_copy(dst_hbm.at[s], dst_v)
    pltpu.sync_copy(idn_hbm, idn_v)

    def z_agg(g, carry):
        agg_v[g // (DH // L), pl.ds(lax.rem(g, DH // L) * L, L)] = zeros
        return carry

    def z_hist(g, carry):
        hist_v[g, :] = zeros
        return carry

    lax.fori_loop(0, ZR * DH // L, z_agg, 0)
    lax.fori_loop(0, HR, z_hist, 0)
    for p in range(3):
        pltpu.sync_copy(agg_v, agg_s.at[pl.ds((3 * s + p) * ZR, ZR)])

    @pl.when(s == NS - 1)
    def _():
        pltpu.sync_copy(agg_v.at[pl.ds(0, N - 48 * ZR)],
                        agg_s.at[pl.ds(48 * ZR, N - 48 * ZR)])

    pltpu.sync_copy(hist_v.at[pl.ds(0, HR // NS)],
                    deg_s.at[pl.ds(s * (HR // NS), HR // NS)])
    plsc.subcore_barrier()

    for b in range(NB - 1):
        pltpu.async_copy(xh.at[src_v.at[b]], rows_v.at[b], gsem)

    def body(i, carry):
        b = lax.rem(i, NB)

        @pl.when(i >= 1)
        def _():
            pltpu.make_async_copy(rows_v.at[0], agg_s.at[dst_v.at[0]],
                                  ssem).wait()

        @pl.when(i + NB - 1 < NCHUNK)
        def _():
            nxt = i + NB - 1
            pltpu.async_copy(xh.at[src_v.at[nxt]], rows_v.at[lax.rem(nxt, NB)],
                             gsem)

        for g in range(K // L):
            idx = dst_v[i, pl.ds(g * L, L)]
            plsc.addupdate_scatter(
                hist_v,
                [lax.shift_right_logical(idx, 4), lax.bitwise_and(idx, 15)],
                ones)

        pltpu.make_async_copy(xh.at[src_v.at[i]], rows_v.at[b], gsem).wait()
        pltpu.async_copy(rows_v.at[b], agg_s.at[dst_v.at[i]], ssem, add=True)
        return carry

    lax.fori_loop(0, NCHUNK, body, 0)

    pltpu.make_async_copy(rows_v.at[0], agg_s.at[dst_v.at[0]], ssem).wait()
    for j in range(HR // 128):
        pltpu.sync_copy(hist_v.at[pl.ds(j * 128, 128)],
                        deg_s.at[idn_v.at[j]], add=True)

    plsc.subcore_barrier()

    def norm_pass(base, rpp):
        pltpu.sync_copy(agg_s.at[pl.ds(base, rpp)], agg_v.at[pl.ds(0, rpp)])
        pltpu.sync_copy(deg_s.at[pl.ds(base // L, rpp // L)],
                        degm_v.at[pl.ds(0, rpp // L)])

        def gbody(g, carry):
            inv = 1.0 / jnp.maximum(degm_v[g, :], 1.0)
            for l in range(L):
                r = g * L + l
                for q in range(DH // L):
                    agg_v[r, pl.ds(q * L, L)] = (
                        agg_v[r, pl.ds(q * L, L)] * inv[l])
            return carry

        lax.fori_loop(0, rpp // L, gbody, 0)
        pltpu.sync_copy(agg_v.at[pl.ds(0, rpp)],
                        y_hbm.at[pl.ds(base, rpp), pl.ds(c * DH, DH)])

    @pl.when(s < NS - 1)
    def _():
        for p in range(3):
            norm_pass(s * 624 + p * ZR, ZR)

    @pl.when(s == NS - 1)
    def _():
        for p in range(4):
            norm_pass(15 * 624 + p * 160, 160)


def kernel(x, edge_index):
    ei = edge_index.astype(jnp.int32)
    src = ei[0].reshape(NS, NCHUNK, K)
    dst = ei[1].reshape(NS, NCHUNK, K)
    x_halves = jnp.stack([x[:, :DH], x[:, DH:]])
    idn = jnp.arange(HR, dtype=jnp.int32).reshape(HR // 128, 128)
    return _sc_smooth(x_halves, src, dst, idn)


# speedup vs baseline: 15.5431x; 1.0559x over previous
import functools

import jax
import jax.numpy as jnp
from jax import lax
from jax.experimental import pallas as pl
from jax.experimental.pallas import tpu as pltpu
from jax.experimental.pallas import tpu_sc as plsc

N = 10000
D = 128
DH = D // 2
E = 320000
NC, NS = 2, 16
EPT = E // NS
K = 80
NCHUNK = EPT // K
NB = 4
L = 16
HR = 640
ZR = 208

_mesh = plsc.VectorSubcoreMesh(core_axis_name="c", subcore_axis_name="s")


@functools.partial(
    pl.kernel,
    mesh=_mesh,
    compiler_params=pltpu.CompilerParams(use_tc_tiling_on_sc=False,
                                         needs_layout_passes=False),
    out_type=[
        jax.ShapeDtypeStruct((N, D), jnp.float32),
        jax.ShapeDtypeStruct((NC, N, DH), jnp.float32),
    ],
    scratch_types=[
        pltpu.VMEM((NCHUNK, K), jnp.int32),
        pltpu.VMEM((NCHUNK, K), jnp.int32),
        pltpu.VMEM((NB, K, DH), jnp.float32),
        pltpu.VMEM((ZR, DH), jnp.float32),
        pltpu.VMEM((HR, L), jnp.float32),
        pltpu.VMEM((HR // L, L), jnp.float32),
        pltpu.VMEM((HR // 128, 128), jnp.int32),
        pltpu.VMEM_SHARED((N, DH), jnp.float32),
        pltpu.VMEM_SHARED((HR, L), jnp.float32),
        pltpu.SemaphoreType.DMA,
        pltpu.SemaphoreType.DMA,
    ],
)
def _sc_smooth(x_hbm, src_hbm, dst_hbm, idn_hbm,
               y_hbm, xh_hbm, src_v, dst_v, rows_v, agg_v, hist_v, degm_v,
               idn_v, agg_s, deg_s, gsem, ssem):
    c = lax.axis_index("c")
    s = lax.axis_index("s")
    xh = xh_hbm.at[c]
    ones = jnp.full((L,), 1.0, jnp.float32)
    zeros = jnp.zeros((L,), jnp.float32)

    pltpu.sync_copy(src_hbm.at[s], src_v)
    pltpu.sync_copy(dst_hbm.at[s], dst_v)
    pltpu.sync_copy(idn_hbm, idn_v)

    def z_agg(g, carry):
        agg_v[g // (DH // L), pl.ds(lax.rem(g, DH // L) * L, L)] = zeros
        return carry

    def z_hist(g, carry):
        hist_v[g, :] = zeros
        return carry

    lax.fori_loop(0, ZR * DH // L, z_agg, 0)
    lax.fori_loop(0, HR, z_hist, 0)
    for p in range(3):
        pltpu.sync_copy(agg_v, agg_s.at[pl.ds((3 * s + p) * ZR, ZR)])

    @pl.when(s == NS - 1)
    def _():
        pltpu.sync_copy(agg_v.at[pl.ds(0, N - 48 * ZR)],
                        agg_s.at[pl.ds(48 * ZR, N - 48 * ZR)])

    pltpu.sync_copy(hist_v.at[pl.ds(0, HR // NS)],
                    deg_s.at[pl.ds(s * (HR // NS), HR // NS)])

    for p in range(5):
        row0 = s * 625 + p * 125
        pltpu.sync_copy(x_hbm.at[pl.ds(row0, 125), pl.ds(c * DH, DH)],
                        agg_v.at[pl.ds(0, 125)])
        pltpu.sync_copy(agg_v.at[pl.ds(0, 125)], xh.at[pl.ds(row0, 125)])

    plsc.subcore_barrier()

    for b in range(NB - 1):
        pltpu.async_copy(xh.at[src_v.at[b]], rows_v.at[b], gsem)

    def body(i, carry):
        b = lax.rem(i, NB)

        @pl.when(i >= 1)
        def _():
            pltpu.make_async_copy(rows_v.at[0], agg_s.at[dst_v.at[0]],
                                  ssem).wait()

        @pl.when(i + NB - 1 < NCHUNK)
        def _():
            nxt = i + NB - 1
            pltpu.async_copy(xh.at[src_v.at[nxt]], rows_v.at[lax.rem(nxt, NB)],
                             gsem)

        for g in range(K // L):
            idx = dst_v[i, pl.ds(g * L, L)]
            plsc.addupdate_scatter(
                hist_v,
                [lax.shift_right_logical(idx, 4), lax.bitwise_and(idx, 15)],
                ones)

        pltpu.make_async_copy(xh.at[src_v.at[i]], rows_v.at[b], gsem).wait()
        pltpu.async_copy(rows_v.at[b], agg_s.at[dst_v.at[i]], ssem, add=True)
        return carry

    lax.fori_loop(0, NCHUNK, body, 0)

    pltpu.make_async_copy(rows_v.at[0], agg_s.at[dst_v.at[0]], ssem).wait()
    for j in range(HR // 128):
        pltpu.sync_copy(hist_v.at[pl.ds(j * 128, 128)],
                        deg_s.at[idn_v.at[j]], add=True)

    plsc.subcore_barrier()

    def norm_pass(base, rpp):
        pltpu.sync_copy(agg_s.at[pl.ds(base, rpp)], agg_v.at[pl.ds(0, rpp)])
        pltpu.sync_copy(deg_s.at[pl.ds(base // L, rpp // L)],
                        degm_v.at[pl.ds(0, rpp // L)])

        def gbody(g, carry):
            inv = 1.0 / jnp.maximum(degm_v[g, :], 1.0)
            for l in range(L):
                r = g * L + l
                for q in range(DH // L):
                    agg_v[r, pl.ds(q * L, L)] = (
                        agg_v[r, pl.ds(q * L, L)] * inv[l])
            return carry

        lax.fori_loop(0, rpp // L, gbody, 0)
        pltpu.sync_copy(agg_v.at[pl.ds(0, rpp)],
                        y_hbm.at[pl.ds(base, rpp), pl.ds(c * DH, DH)])

    @pl.when(s < NS - 1)
    def _():
        for p in range(3):
            norm_pass(s * 624 + p * ZR, ZR)

    @pl.when(s == NS - 1)
    def _():
        for p in range(4):
            norm_pass(15 * 624 + p * 160, 160)


def kernel(x, edge_index):
    ei = edge_index.astype(jnp.int32)
    src = ei[0].reshape(NS, NCHUNK, K)
    dst = ei[1].reshape(NS, NCHUNK, K)
    idn = jnp.arange(HR, dtype=jnp.int32).reshape(HR // 128, 128)
    y, _ = _sc_smooth(x, src, dst, idn)
    return y
